# baseline (device time: 12960 ns/iter reference)
import jax
import jax.numpy as jnp
from jax import lax
from jax.experimental import pallas as pl
from jax.experimental.pallas import tpu as pltpu

N_DEV = 4
BLOCK_ORDER = (2, 1, 3, 0)


def _gelu(y):
    c = 0.7978845608028654
    return 0.5 * y * (1.0 + jnp.tanh(c * (y + 0.044715 * y * y * y)))


def kernel(x, w_mat):
    m_per, k = x.shape
    n = w_mat.shape[1]
    n_per = n // N_DEV

    def body(x_hbm, w_hbm, out_hbm, xv, wv, zs, own,
             load_sems, out_sem, send_sems, recv_sems):
        my = lax.axis_index("i")

        barrier_sem = pltpu.get_barrier_semaphore()
        for off in range(1, N_DEV):
            peer = (my + off) % N_DEV
            pl.semaphore_signal(
                barrier_sem, inc=1,
                device_id=(peer,), device_id_type=pl.DeviceIdType.MESH,
            )

        cp_x = pltpu.make_async_copy(x_hbm, xv, load_sems.at[N_DEV])
        cp_x.start()
        wcps = []
        for i, off in enumerate(BLOCK_ORDER):
            tgt = (my + off) % N_DEV
            cp = pltpu.make_async_copy(
                w_hbm.at[:, pl.ds(tgt * n_per, n_per)],
                wv.at[i],
                load_sems.at[i],
            )
            cp.start()
            wcps.append(cp)

        cp_x.wait()
        xb = xv[...].astype(jnp.bfloat16)

        rdmas = []
        cp_out = None
        for i, off in enumerate(BLOCK_ORDER):
            wcps[i].wait()
            wb = wv[i].astype(jnp.bfloat16)
            blk = _gelu(jnp.dot(xb, wb, preferred_element_type=jnp.float32))
            if off == 0:
                own[...] = blk.astype(jnp.bfloat16)
                cp_out = pltpu.make_async_copy(
                    own, out_hbm.at[pl.ds(my * m_per, m_per), :], out_sem
                )
                cp_out.start()
                continue
            zs[off - 1] = blk.astype(jnp.bfloat16)
            if i == 0:
                pl.semaphore_wait(barrier_sem, N_DEV - 1)
            rdma = pltpu.make_async_remote_copy(
                src_ref=zs.at[off - 1],
                dst_ref=out_hbm.at[pl.ds(my * m_per, m_per), :],
                send_sem=send_sems.at[off - 1],
                recv_sem=recv_sems.at[off - 1],
                device_id=((my + off) % N_DEV,),
                device_id_type=pl.DeviceIdType.MESH,
            )
            rdma.start()
            rdmas.append(rdma)

        for off in range(1, N_DEV):
            src = (my - off) % N_DEV
            recv = pltpu.make_async_remote_copy(
                src_ref=zs.at[off - 1],
                dst_ref=out_hbm.at[pl.ds(src * m_per, m_per), :],
                send_sem=send_sems.at[off - 1],
                recv_sem=recv_sems.at[off - 1],
                device_id=(src,),
                device_id_type=pl.DeviceIdType.MESH,
            )
            recv.wait_recv()
        for rdma in rdmas:
            rdma.wait_send()
        cp_out.wait()

    out_shape = jax.ShapeDtypeStruct((N_DEV * m_per, n_per), jnp.bfloat16)
    return pl.pallas_call(
        body,
        out_shape=out_shape,
        in_specs=[
            pl.BlockSpec(memory_space=pltpu.MemorySpace.HBM),
            pl.BlockSpec(memory_space=pltpu.MemorySpace.HBM),
        ],
        out_specs=pl.BlockSpec(memory_space=pltpu.MemorySpace.HBM),
        scratch_shapes=[
            pltpu.VMEM((m_per, k), jnp.float32),
            pltpu.VMEM((N_DEV, k, n_per), jnp.float32),
            pltpu.VMEM((N_DEV - 1, m_per, n_per), jnp.bfloat16),
            pltpu.VMEM((m_per, n_per), jnp.bfloat16),
            pltpu.SemaphoreType.DMA((N_DEV + 1,)),
            pltpu.SemaphoreType.DMA,
            pltpu.SemaphoreType.DMA((N_DEV - 1,)),
            pltpu.SemaphoreType.DMA((N_DEV - 1,)),
        ],
        compiler_params=pltpu.CompilerParams(collective_id=0),
    )(x, w_mat)


# device time: 12585 ns/iter; 1.0298x vs baseline; 1.0298x over previous
import jax
import jax.numpy as jnp
from jax import lax
from jax.experimental import pallas as pl
from jax.experimental.pallas import tpu as pltpu

N_DEV = 4
SEND_ORDER = (2, 1, 3)


def _gelu(y):
    c = 0.7978845608028654
    return 0.5 * y * (1.0 + jnp.tanh(c * (y + 0.044715 * y * y * y)))


def kernel(x, w_mat):
    m_per, k = x.shape
    n = w_mat.shape[1]
    n_per = n // N_DEV

    def body(x_ref, w_ref, out_ref, z_ref, send_sems, recv_sems):
        my = lax.axis_index("i")

        barrier_sem = pltpu.get_barrier_semaphore()
        for off in range(1, N_DEV):
            peer = (my + off) % N_DEV
            pl.semaphore_signal(
                barrier_sem, inc=1,
                device_id=(peer,), device_id_type=pl.DeviceIdType.MESH,
            )

        xb = x_ref[...]

        rdmas = []
        for i, off in enumerate(SEND_ORDER):
            tgt = (my + off) % N_DEV
            wb = w_ref[:, pl.ds(tgt * n_per, n_per)]
            blk = _gelu(jnp.dot(xb, wb, preferred_element_type=jnp.float32))
            z_ref[off - 1] = blk.astype(jnp.bfloat16)
            if i == 0:
                pl.semaphore_wait(barrier_sem, N_DEV - 1)
            rdma = pltpu.make_async_remote_copy(
                src_ref=z_ref.at[off - 1],
                dst_ref=out_ref.at[pl.ds(my * m_per, m_per), :],
                send_sem=send_sems.at[off - 1],
                recv_sem=recv_sems.at[off - 1],
                device_id=(tgt,),
                device_id_type=pl.DeviceIdType.MESH,
            )
            rdma.start()
            rdmas.append(rdma)

        wb = w_ref[:, pl.ds(my * n_per, n_per)]
        blk = _gelu(jnp.dot(xb, wb, preferred_element_type=jnp.float32))
        out_ref[pl.ds(my * m_per, m_per), :] = blk.astype(jnp.bfloat16)

        for off in range(1, N_DEV):
            src = (my - off) % N_DEV
            recv = pltpu.make_async_remote_copy(
                src_ref=z_ref.at[off - 1],
                dst_ref=out_ref.at[pl.ds(src * m_per, m_per), :],
                send_sem=send_sems.at[off - 1],
                recv_sem=recv_sems.at[off - 1],
                device_id=(src,),
                device_id_type=pl.DeviceIdType.MESH,
            )
            recv.wait_recv()
        for rdma in rdmas:
            rdma.wait_send()

    out_shape = jax.ShapeDtypeStruct((N_DEV * m_per, n_per), jnp.bfloat16)
    run = pl.pallas_call(
        body,
        out_shape=out_shape,
        in_specs=[
            pl.BlockSpec(memory_space=pltpu.VMEM),
            pl.BlockSpec(memory_space=pltpu.VMEM),
        ],
        out_specs=pl.BlockSpec(memory_space=pltpu.VMEM),
        scratch_shapes=[
            pltpu.VMEM((N_DEV - 1, m_per, n_per), jnp.bfloat16),
            pltpu.SemaphoreType.DMA((N_DEV - 1,)),
            pltpu.SemaphoreType.DMA((N_DEV - 1,)),
        ],
        compiler_params=pltpu.CompilerParams(collective_id=0),
    )
    return run(x.astype(jnp.bfloat16), w_mat.astype(jnp.bfloat16))


# device time: 10369 ns/iter; 1.2499x vs baseline; 1.2137x over previous
import jax
import jax.numpy as jnp
from jax import lax
from jax.experimental import pallas as pl
from jax.experimental.pallas import tpu as pltpu

N_DEV = 4
BLOCK_ORDER = (2, 1, 3, 0)


def _gelu(y):
    c = 0.7978845608028654
    return 0.5 * y * (1.0 + jnp.tanh(c * (y + 0.044715 * y * y * y)))


def kernel(x, w_mat):
    m_per, k = x.shape
    n = w_mat.shape[1]
    n_per = n // N_DEV

    def body(x_hbm, w_hbm, out_ref, xv, wv, zs,
             load_sems, send_sems, recv_sems):
        my = lax.axis_index("i")

        barrier_sem = pltpu.get_barrier_semaphore()
        for off in range(1, N_DEV):
            peer = (my + off) % N_DEV
            pl.semaphore_signal(
                barrier_sem, inc=1,
                device_id=(peer,), device_id_type=pl.DeviceIdType.MESH,
            )

        cp_x = pltpu.make_async_copy(x_hbm, xv, load_sems.at[N_DEV])
        cp_x.start()
        wcps = []
        for i, off in enumerate(BLOCK_ORDER):
            tgt = (my + off) % N_DEV
            cp = pltpu.make_async_copy(
                w_hbm.at[:, pl.ds(tgt * n_per, n_per)],
                wv.at[i],
                load_sems.at[i],
            )
            cp.start()
            wcps.append(cp)

        cp_x.wait()
        xb = xv[...].astype(jnp.bfloat16)

        rdmas = []
        for i, off in enumerate(BLOCK_ORDER):
            wcps[i].wait()
            wb = wv[i].astype(jnp.bfloat16)
            blk = _gelu(jnp.dot(xb, wb, preferred_element_type=jnp.float32))
            if off == 0:
                out_ref[pl.ds(my * m_per, m_per), :] = blk.astype(jnp.bfloat16)
                continue
            zs[off - 1] = blk.astype(jnp.bfloat16)
            if i == 0:
                pl.semaphore_wait(barrier_sem, N_DEV - 1)
            rdma = pltpu.make_async_remote_copy(
                src_ref=zs.at[off - 1],
                dst_ref=out_ref.at[pl.ds(my * m_per, m_per), :],
                send_sem=send_sems.at[off - 1],
                recv_sem=recv_sems.at[off - 1],
                device_id=((my + off) % N_DEV,),
                device_id_type=pl.DeviceIdType.MESH,
            )
            rdma.start()
            rdmas.append(rdma)

        for off in range(1, N_DEV):
            src = (my - off) % N_DEV
            recv = pltpu.make_async_remote_copy(
                src_ref=zs.at[off - 1],
                dst_ref=out_ref.at[pl.ds(src * m_per, m_per), :],
                send_sem=send_sems.at[off - 1],
                recv_sem=recv_sems.at[off - 1],
                device_id=(src,),
                device_id_type=pl.DeviceIdType.MESH,
            )
            recv.wait_recv()
        for rdma in rdmas:
            rdma.wait_send()

    out_shape = jax.ShapeDtypeStruct((N_DEV * m_per, n_per), jnp.bfloat16)
    run = pl.pallas_call(
        body,
        out_shape=out_shape,
        in_specs=[
            pl.BlockSpec(memory_space=pltpu.MemorySpace.HBM),
            pl.BlockSpec(memory_space=pltpu.MemorySpace.HBM),
        ],
        out_specs=pl.BlockSpec(memory_space=pltpu.VMEM),
        scratch_shapes=[
            pltpu.VMEM((m_per, k), jnp.float32),
            pltpu.VMEM((N_DEV, k, n_per), jnp.float32),
            pltpu.VMEM((N_DEV - 1, m_per, n_per), jnp.bfloat16),
            pltpu.SemaphoreType.DMA((N_DEV + 1,)),
            pltpu.SemaphoreType.DMA((N_DEV - 1,)),
            pltpu.SemaphoreType.DMA((N_DEV - 1,)),
        ],
        compiler_params=pltpu.CompilerParams(collective_id=0),
    )
    return run(
        pltpu.with_memory_space_constraint(x, pltpu.MemorySpace.HBM),
        pltpu.with_memory_space_constraint(w_mat, pltpu.MemorySpace.HBM),
    )
